# baseline (speedup 1.0000x reference)
import functools

import jax
import jax.numpy as jnp
from jax import lax
from jax.experimental import pallas as pl
from jax.experimental.pallas import tpu as pltpu
from jax.experimental.pallas import tpu_sc as plsc

N, C, D, H, W = 6, 80, 118, 16, 44
DHW = D * H * W
V = N * DHW
BEV_H, BEV_W = 128, 128
K = BEV_H * BEV_W
PTS = 16
M = K * PTS

NC, NS, L = 2, 16, 16
NW = NC * NS
KW = K // NW
G = 8
PG = G * PTS
NCHUNK = KW // G
CB = C // L

TBLK = 7552
NB = DHW // TBLK
NSTEP = N * NB


CP = 128


def _start_in_dmas(cf_any, inb, isem, n, dst_slot):
    for c in range(C):
        pltpu.make_async_copy(
            cf_any.at[pl.ds((n * C + c) * DHW, DHW)],
            inb.at[dst_slot, c], isem.at[dst_slot],
        ).start()


def _wait_in_dmas(cf_any, inb, isem, slot):
    for c in range(C):
        pltpu.make_async_copy(
            cf_any.at[pl.ds(c * DHW, DHW)], inb.at[slot, c], isem.at[slot]
        ).wait()


def _tr_in_body(cf_any, out_any, inb, outb, isem, osem):
    n = pl.program_id(0)
    slot = lax.rem(n, 2)
    nxt = lax.rem(n + 1, 2)

    def _out_copy(o, s):
        return pltpu.make_async_copy(
            outb.at[s], out_any.at[pl.ds(o * TBLK, TBLK), :], osem.at[s]
        )

    @pl.when(n == 0)
    def _prologue():
        _start_in_dmas(cf_any, inb, isem, n, slot)

    @pl.when(n + 1 < N)
    def _prefetch():
        _start_in_dmas(cf_any, inb, isem, n + 1, nxt)

    _wait_in_dmas(cf_any, inb, isem, slot)

    for b in range(NB):
        o = n * NB + b
        os = b % 2

        @pl.when(o >= 2)
        def _drain(o=o, os=os):
            _out_copy(o, os).wait()

        outb[os, :, 0:C] = inb[slot, :, pl.ds(b * TBLK, TBLK)].T
        _out_copy(o, os).start()

    @pl.when(n == N - 1)
    def _epilogue():
        _out_copy(N * NB - 2, (NB - 2) % 2).wait()
        _out_copy(N * NB - 1, (NB - 1) % 2).wait()


def _transpose_features(cf_flat):
    return pl.pallas_call(
        _tr_in_body,
        grid=(N,),
        in_specs=[pl.BlockSpec(memory_space=pl.ANY)],
        out_specs=pl.BlockSpec(memory_space=pl.ANY),
        out_shape=jax.ShapeDtypeStruct((V, CP), jnp.float32),
        scratch_shapes=[
            pltpu.VMEM((2, C, DHW), jnp.float32),
            pltpu.VMEM((2, TBLK, CP), jnp.float32),
            pltpu.SemaphoreType.DMA((2,)),
            pltpu.SemaphoreType.DMA((2,)),
        ],
        compiler_params=pltpu.CompilerParams(
            vmem_limit_bytes=120 * 1024 * 1024,
        ),
    )(cf_flat)


@functools.cache
def _get_sc_pool():
    mesh = plsc.VectorSubcoreMesh(
        core_axis_name="c", subcore_axis_name="s", num_cores=NC, num_subcores=NS
    )

    @functools.partial(
        pl.kernel,
        out_type=jax.ShapeDtypeStruct((K, CP), jnp.float32),
        mesh=mesh,
        compiler_params=pltpu.CompilerParams(
            needs_layout_passes=False, use_tc_tiling_on_sc=False
        ),
        scratch_types=[
            pltpu.VMEM((PG,), jnp.int32),
            pltpu.VMEM((PG, CP), jnp.float32),
            pltpu.VMEM((PG,), jnp.float32),
            pltpu.VMEM((G, CP), jnp.float32),
            pltpu.VMEM((G,), jnp.int32),
            pltpu.SemaphoreType.DMA,
            pltpu.SemaphoreType.DMA,
            pltpu.SemaphoreType.DMA,
        ],
    )
    def _sc_pool(idx_hbm, dw_hbm, feat_hbm, bev_hbm, out_hbm,
                 idx_v, rows_v, w_v, acc_v, bev_v, gsem, wsem, ssem):
        wid = lax.axis_index("s") * NC + lax.axis_index("c")
        k0 = wid * KW

        def chunk_body(ch, carry):
            kbase = k0 + ch * G
            pbase = kbase * PTS
            pltpu.sync_copy(idx_hbm.at[pl.ds(pbase, PG)], idx_v)
            gcopy = pltpu.async_copy(feat_hbm.at[idx_v], rows_v, gsem)
            wcopy = pltpu.async_copy(dw_hbm.at[idx_v], w_v, wsem)
            pltpu.sync_copy(bev_hbm.at[pl.ds(kbase, G)], bev_v)
            gcopy.wait()
            wcopy.wait()

            @pl.loop(0, G)
            def _intervals(g):
                accs = None
                for j in range(PTS):
                    p = g * PTS + j
                    bw = plsc.load_gather(w_v, [jnp.full((L,), p, jnp.int32)])
                    terms = [
                        bw * rows_v[p, pl.ds(cb * L, L)] for cb in range(CB)
                    ]
                    if accs is None:
                        accs = terms
                    else:
                        accs = [a + t for a, t in zip(accs, terms)]
                for cb in range(CB):
                    acc_v[g, pl.ds(cb * L, L)] = accs[cb]

            pltpu.async_copy(acc_v, out_hbm.at[bev_v], ssem).wait()
            return carry

        lax.fori_loop(0, NCHUNK, chunk_body, 0)

    return _sc_pool


OB = 2048


def _tr_out_body(x_ref, o_ref):
    o_ref[...] = x_ref[:, 0:C].T


def _transpose_out(rows):
    return pl.pallas_call(
        _tr_out_body,
        grid=(K // OB,),
        in_specs=[pl.BlockSpec((OB, CP), lambda b: (b, 0))],
        out_specs=pl.BlockSpec((C, OB), lambda b: (0, b)),
        out_shape=jax.ShapeDtypeStruct((C, K), jnp.float32),
    )(rows)


def kernel(camera_features, depth_weights, indices, intervals):
    feat_pad = _transpose_features(camera_features)
    bev = intervals[:, 2].astype(jnp.int32)
    rows = _get_sc_pool()(indices.astype(jnp.int32), depth_weights, feat_pad, bev)
    out = _transpose_out(rows)
    return out.reshape(1, C, BEV_H, BEV_W)


# speedup vs baseline: 1.0566x; 1.0566x over previous
import functools

import jax
import jax.numpy as jnp
from jax import lax
from jax.experimental import pallas as pl
from jax.experimental.pallas import tpu as pltpu
from jax.experimental.pallas import tpu_sc as plsc

N, C, D, H, W = 6, 80, 118, 16, 44
DHW = D * H * W
V = N * DHW
BEV_H, BEV_W = 128, 128
K = BEV_H * BEV_W
PTS = 16
M = K * PTS

NC, NS, L = 2, 16, 16
NW = NC * NS
KW = K // NW
G = 32
PG = G * PTS
PR = PG // 128
NCHUNK = KW // G
CB = C // L

TBLK = 7552
NB = DHW // TBLK
NSTEP = N * NB


CP = 128


def _start_in_dmas(cf_any, inb, isem, n, dst_slot):
    for c in range(C):
        pltpu.make_async_copy(
            cf_any.at[pl.ds((n * C + c) * DHW, DHW)],
            inb.at[dst_slot, c], isem.at[dst_slot],
        ).start()


def _wait_in_dmas(cf_any, inb, isem, slot):
    for c in range(C):
        pltpu.make_async_copy(
            cf_any.at[pl.ds(c * DHW, DHW)], inb.at[slot, c], isem.at[slot]
        ).wait()


def _tr_in_body(cf_any, out_any, inb, outb, isem, osem):
    n = pl.program_id(0)
    slot = lax.rem(n, 2)
    nxt = lax.rem(n + 1, 2)

    def _out_copy(o, s):
        return pltpu.make_async_copy(
            outb.at[s], out_any.at[pl.ds(o * TBLK, TBLK), :], osem.at[s]
        )

    @pl.when(n == 0)
    def _prologue():
        _start_in_dmas(cf_any, inb, isem, n, slot)

    @pl.when(n + 1 < N)
    def _prefetch():
        _start_in_dmas(cf_any, inb, isem, n + 1, nxt)

    _wait_in_dmas(cf_any, inb, isem, slot)

    for b in range(NB):
        o = n * NB + b
        os = b % 2

        @pl.when(o >= 2)
        def _drain(o=o, os=os):
            _out_copy(o, os).wait()

        outb[os, :, 0:C] = inb[slot, :, pl.ds(b * TBLK, TBLK)].T
        _out_copy(o, os).start()

    @pl.when(n == N - 1)
    def _epilogue():
        _out_copy(N * NB - 2, (NB - 2) % 2).wait()
        _out_copy(N * NB - 1, (NB - 1) % 2).wait()


def _transpose_features(cf_flat):
    return pl.pallas_call(
        _tr_in_body,
        grid=(N,),
        in_specs=[pl.BlockSpec(memory_space=pl.ANY)],
        out_specs=pl.BlockSpec(memory_space=pl.ANY),
        out_shape=jax.ShapeDtypeStruct((V, CP), jnp.float32),
        scratch_shapes=[
            pltpu.VMEM((2, C, DHW), jnp.float32),
            pltpu.VMEM((2, TBLK, CP), jnp.float32),
            pltpu.SemaphoreType.DMA((2,)),
            pltpu.SemaphoreType.DMA((2,)),
        ],
        compiler_params=pltpu.CompilerParams(
            vmem_limit_bytes=120 * 1024 * 1024,
        ),
    )(cf_flat)


@functools.cache
def _get_sc_pool():
    mesh = plsc.VectorSubcoreMesh(
        core_axis_name="c", subcore_axis_name="s", num_cores=NC, num_subcores=NS
    )

    @functools.partial(
        pl.kernel,
        out_type=jax.ShapeDtypeStruct((K, CP), jnp.float32),
        mesh=mesh,
        compiler_params=pltpu.CompilerParams(
            needs_layout_passes=False, use_tc_tiling_on_sc=False
        ),
        scratch_types=[
            pltpu.VMEM((PG,), jnp.int32),
            pltpu.VMEM((PG, CP), jnp.float32),
            pltpu.VMEM((PG,), jnp.float32),
            pltpu.VMEM((G, CP), jnp.float32),
            pltpu.VMEM((G,), jnp.int32),
            pltpu.SemaphoreType.DMA,
            pltpu.SemaphoreType.DMA,
            pltpu.SemaphoreType.DMA,
        ],
    )
    def _sc_pool(idx_hbm, dw_hbm, feat_hbm, bev_hbm, out_hbm,
                 idx_v, rows_v, w_v, acc_v, bev_v, gsem, wsem, ssem):
        wid = lax.axis_index("s") * NC + lax.axis_index("c")
        k0 = wid * KW

        def chunk_body(ch, carry):
            kbase = k0 + ch * G
            pbase = kbase * PTS
            pltpu.sync_copy(idx_hbm.at[pl.ds(pbase, PG)], idx_v)
            pltpu.sync_copy(bev_hbm.at[pl.ds(kbase, G)], bev_v)
            for j in range(PR):
                gcopy = pltpu.async_copy(
                    feat_hbm.at[idx_v.at[pl.ds(j * 128, 128)]],
                    rows_v.at[pl.ds(j * 128, 128), :], gsem,
                )
                wcopy = pltpu.async_copy(
                    dw_hbm.at[idx_v.at[pl.ds(j * 128, 128)]],
                    w_v.at[pl.ds(j * 128, 128)], wsem,
                )
                gcopy.wait()
                wcopy.wait()

            @pl.loop(0, G)
            def _intervals(g):
                accs = None
                for j in range(PTS):
                    p = g * PTS + j
                    bw = plsc.load_gather(w_v, [jnp.full((L,), p, jnp.int32)])
                    terms = [
                        bw * rows_v[p, pl.ds(cb * L, L)] for cb in range(CB)
                    ]
                    if accs is None:
                        accs = terms
                    else:
                        accs = [a + t for a, t in zip(accs, terms)]
                for cb in range(CB):
                    acc_v[g, pl.ds(cb * L, L)] = accs[cb]

            pltpu.async_copy(acc_v, out_hbm.at[bev_v], ssem).wait()
            return carry

        lax.fori_loop(0, NCHUNK, chunk_body, 0)

    return _sc_pool


OB = 2048


def _tr_out_body(x_ref, o_ref):
    o_ref[...] = x_ref[:, 0:C].T


def _transpose_out(rows):
    return pl.pallas_call(
        _tr_out_body,
        grid=(K // OB,),
        in_specs=[pl.BlockSpec((OB, CP), lambda b: (b, 0))],
        out_specs=pl.BlockSpec((C, OB), lambda b: (0, b)),
        out_shape=jax.ShapeDtypeStruct((C, K), jnp.float32),
    )(rows)


def kernel(camera_features, depth_weights, indices, intervals):
    feat_pad = _transpose_features(camera_features)
    bev = intervals[:, 2].astype(jnp.int32)
    rows = _get_sc_pool()(indices.astype(jnp.int32), depth_weights, feat_pad, bev)
    out = _transpose_out(rows)
    return out.reshape(1, C, BEV_H, BEV_W)


# speedup vs baseline: 1.1342x; 1.0734x over previous
import functools

import jax
import jax.numpy as jnp
from jax import lax
from jax.experimental import pallas as pl
from jax.experimental.pallas import tpu as pltpu
from jax.experimental.pallas import tpu_sc as plsc

N, C, D, H, W = 6, 80, 118, 16, 44
DHW = D * H * W
V = N * DHW
BEV_H, BEV_W = 128, 128
K = BEV_H * BEV_W
PTS = 16
M = K * PTS

NC, NS, L = 2, 16, 16
NW = NC * NS
KW = K // NW
G = 32
PG = G * PTS
PR = PG // 128
NCHUNK = KW // G
CB = C // L

TBLK = 7552
NB = DHW // TBLK
NSTEP = N * NB


CP = 128


def _start_in_dmas(cf_any, inb, isem, n, dst_slot):
    for c in range(C):
        pltpu.make_async_copy(
            cf_any.at[pl.ds((n * C + c) * DHW, DHW)],
            inb.at[dst_slot, c], isem.at[dst_slot],
        ).start()


def _wait_in_dmas(cf_any, inb, isem, slot):
    for c in range(C):
        pltpu.make_async_copy(
            cf_any.at[pl.ds(c * DHW, DHW)], inb.at[slot, c], isem.at[slot]
        ).wait()


def _tr_in_body(cf_any, out_any, inb, outb, isem, osem):
    n = pl.program_id(0)
    slot = lax.rem(n, 2)
    nxt = lax.rem(n + 1, 2)

    def _out_copy(o, s):
        return pltpu.make_async_copy(
            outb.at[s], out_any.at[pl.ds(o * TBLK, TBLK), :], osem.at[s]
        )

    @pl.when(n == 0)
    def _prologue():
        _start_in_dmas(cf_any, inb, isem, n, slot)

    @pl.when(n + 1 < N)
    def _prefetch():
        _start_in_dmas(cf_any, inb, isem, n + 1, nxt)

    _wait_in_dmas(cf_any, inb, isem, slot)

    for b in range(NB):
        o = n * NB + b
        os = b % 2

        @pl.when(o >= 2)
        def _drain(o=o, os=os):
            _out_copy(o, os).wait()

        outb[os, :, 0:C] = inb[slot, :, pl.ds(b * TBLK, TBLK)].T
        _out_copy(o, os).start()

    @pl.when(n == N - 1)
    def _epilogue():
        _out_copy(N * NB - 2, (NB - 2) % 2).wait()
        _out_copy(N * NB - 1, (NB - 1) % 2).wait()


def _transpose_features(cf_flat):
    return pl.pallas_call(
        _tr_in_body,
        grid=(N,),
        in_specs=[pl.BlockSpec(memory_space=pl.ANY)],
        out_specs=pl.BlockSpec(memory_space=pl.ANY),
        out_shape=jax.ShapeDtypeStruct((V, CP), jnp.float32),
        scratch_shapes=[
            pltpu.VMEM((2, C, DHW), jnp.float32),
            pltpu.VMEM((2, TBLK, CP), jnp.float32),
            pltpu.SemaphoreType.DMA((2,)),
            pltpu.SemaphoreType.DMA((2,)),
        ],
        compiler_params=pltpu.CompilerParams(
            vmem_limit_bytes=120 * 1024 * 1024,
        ),
    )(cf_flat)


@functools.cache
def _get_sc_pool():
    mesh = plsc.VectorSubcoreMesh(
        core_axis_name="c", subcore_axis_name="s", num_cores=NC, num_subcores=NS
    )

    @functools.partial(
        pl.kernel,
        out_type=jax.ShapeDtypeStruct((K, CP), jnp.float32),
        mesh=mesh,
        compiler_params=pltpu.CompilerParams(
            needs_layout_passes=False, use_tc_tiling_on_sc=False
        ),
        scratch_types=[
            pltpu.VMEM((PG,), jnp.int32),
            pltpu.VMEM((PG, CP), jnp.float32),
            pltpu.VMEM((PG,), jnp.float32),
            pltpu.VMEM((G, CP), jnp.float32),
            pltpu.VMEM((G,), jnp.int32),
            pltpu.SemaphoreType.DMA,
            pltpu.SemaphoreType.DMA,
            pltpu.SemaphoreType.DMA,
        ],
    )
    def _sc_pool(idx_hbm, dw_hbm, feat_hbm, bev_hbm, out_hbm,
                 idx_v, rows_v, w_v, acc_v, bev_v, gsem, wsem, ssem):
        wid = lax.axis_index("s") * NC + lax.axis_index("c")
        k0 = wid * KW

        def chunk_body(ch, carry):
            kbase = k0 + ch * G
            pbase = kbase * PTS
            pltpu.sync_copy(idx_hbm.at[pl.ds(pbase, PG)], idx_v)
            pltpu.sync_copy(bev_hbm.at[pl.ds(kbase, G)], bev_v)

            def feat_copy(j):
                return pltpu.make_async_copy(
                    feat_hbm.at[idx_v.at[pl.ds(j * 128, 128)]],
                    rows_v.at[pl.ds(j * 128, 128), :], gsem,
                )

            def w_copy(j):
                return pltpu.make_async_copy(
                    dw_hbm.at[idx_v.at[pl.ds(j * 128, 128)]],
                    w_v.at[pl.ds(j * 128, 128)], wsem,
                )

            feat_copy(0).start()
            w_copy(0).start()
            for j in range(PR):
                w_copy(j).wait()
                if j + 1 < PR:
                    w_copy(j + 1).start()
                feat_copy(j).wait()
                if j + 1 < PR:
                    feat_copy(j + 1).start()

                @pl.loop(j * 8, (j + 1) * 8)
                def _intervals(g):
                    accs = None
                    for t in range(PTS):
                        p = g * PTS + t
                        bw = plsc.load_gather(w_v, [jnp.full((L,), p, jnp.int32)])
                        terms = [
                            bw * rows_v[p, pl.ds(cb * L, L)] for cb in range(CB)
                        ]
                        if accs is None:
                            accs = terms
                        else:
                            accs = [a + t2 for a, t2 in zip(accs, terms)]
                    for cb in range(CB):
                        acc_v[g, pl.ds(cb * L, L)] = accs[cb]

            pltpu.async_copy(acc_v, out_hbm.at[bev_v], ssem).wait()
            return carry

        lax.fori_loop(0, NCHUNK, chunk_body, 0)

    return _sc_pool


OB = 2048


def _tr_out_body(x_ref, o_ref):
    o_ref[...] = x_ref[:, 0:C].T


def _transpose_out(rows):
    return pl.pallas_call(
        _tr_out_body,
        grid=(K // OB,),
        in_specs=[pl.BlockSpec((OB, CP), lambda b: (b, 0))],
        out_specs=pl.BlockSpec((C, OB), lambda b: (0, b)),
        out_shape=jax.ShapeDtypeStruct((C, K), jnp.float32),
    )(rows)


def kernel(camera_features, depth_weights, indices, intervals):
    feat_pad = _transpose_features(camera_features)
    bev = intervals[:, 2].astype(jnp.int32)
    rows = _get_sc_pool()(indices.astype(jnp.int32), depth_weights, feat_pad, bev)
    out = _transpose_out(rows)
    return out.reshape(1, C, BEV_H, BEV_W)
